# Initial kernel scaffold; baseline (speedup 1.0000x reference)
import jax
import jax.numpy as jnp
from jax.experimental import pallas as pl


def kernel(input):
    raise NotImplementedError("write your pallas kernel here")


# speedup vs baseline: 1.1103x; 1.1103x over previous
import functools

import jax
import jax.numpy as jnp
from jax.experimental import pallas as pl
from jax.experimental.pallas import tpu as pltpu

K = 64
ROWS = 8
N = 32768


def _body(x_ref, o_ref, xs_ref):
    xs_ref[...] = x_ref[...]
    iota = jax.lax.broadcasted_iota(jnp.int32, (ROWS, N), 1)
    col_iota = jax.lax.broadcasted_iota(jnp.int32, (ROWS, K), 1)

    def step(k, out):
        x = xs_ref[...]
        m = jnp.max(x, axis=1)
        am = jnp.argmax(x, axis=1)
        xs_ref[...] = jnp.where(iota == am[:, None], -jnp.inf, x)
        return jnp.where(col_iota == k, m[:, None], out)

    out = jax.lax.fori_loop(0, K, step, jnp.full((ROWS, K), -jnp.inf, jnp.float32))
    o_ref[...] = out


@functools.partial(jax.jit, static_argnames=())
def kernel(input):
    rows = input.shape[0]
    grid = rows // ROWS
    return pl.pallas_call(
        _body,
        grid=(grid,),
        in_specs=[pl.BlockSpec((ROWS, N), lambda i: (i, 0))],
        out_specs=pl.BlockSpec((ROWS, K), lambda i: (i, 0)),
        out_shape=jax.ShapeDtypeStruct((rows, K), jnp.float32),
        scratch_shapes=[pltpu.VMEM((ROWS, N), jnp.float32)],
    )(input)


# speedup vs baseline: 3.4900x; 3.1433x over previous
import functools

import jax
import jax.numpy as jnp
from jax import lax
from jax.experimental import pallas as pl
from jax.experimental.pallas import tpu as pltpu
from jax.experimental.pallas import tpu_sc as plsc

K = 64
N = 32768
ROWS = 128
L = 16
VPG = 32
GSZ = VPG * L
G = N // GSZ
NC = 2
NS = 16
NW = NC * NS
RPW = ROWS // NW

def _neg():
    return jnp.full((L,), -jnp.inf, jnp.float32)


def _sortd(v):
    return plsc.sort_key_val(v, v, descending=True)[0]


def _merge2(a, b):
    rb = lax.rev(b, (0,))
    hi = jnp.maximum(a, rb)
    lo = jnp.minimum(a, rb)
    return _sortd(hi), _sortd(lo)


def _insert(s, v):
    s0, s1, s2, s3 = s
    rv = lax.rev(_sortd(v), (0,))
    t3 = _sortd(jnp.maximum(s3, rv))
    s2, t3 = _merge2(s2, t3)
    s1, s2 = _merge2(s1, s2)
    s0, s1 = _merge2(s0, s1)
    return (s0, s1, s2, t3)


def _row_topk(rowbuf, rbuf, obuf):
    def p1(g, s):
        base = g * GSZ
        m = rowbuf[pl.ds(base, L)]
        for j in range(1, VPG):
            m = jnp.maximum(m, rowbuf[pl.ds(base + j * L, L)])
        rbuf[pl.ds(g * L, L)] = m
        return _insert(s, m)

    z = _neg()
    s = lax.fori_loop(0, G, p1, (z, z, z, z))
    t = s[3][15]

    def p2(g, s):
        gmax = rbuf[pl.ds(g * L, L)]

        def scan(ss):
            def body(j, sss):
                v = rowbuf[pl.ds(g * GSZ + j * L, L)]
                return lax.cond(jnp.any(v >= t),
                                lambda q: _insert(q, v), lambda q: q, sss)

            return lax.fori_loop(0, VPG, body, ss)

        return lax.cond(jnp.any(gmax >= t), scan, lambda ss: ss, s)

    s = lax.fori_loop(0, G, p2, (z, z, z, z))
    for b in range(4):
        obuf[pl.ds(b * L, L)] = s[b]


def _make_sc_kernel():
    mesh = plsc.VectorSubcoreMesh(core_axis_name="c", subcore_axis_name="s")

    @functools.partial(
        pl.kernel,
        mesh=mesh,
        out_type=jax.ShapeDtypeStruct((ROWS, K), jnp.float32),
        compiler_params=pltpu.CompilerParams(needs_layout_passes=False),
        scratch_types=[
            pltpu.VMEM((N,), jnp.float32),
            pltpu.VMEM((G * L,), jnp.float32),
            pltpu.VMEM((K,), jnp.float32),
        ],
    )
    def sc_topk(in_hbm, out_hbm, rowbuf, rbuf, obuf):
        wid = lax.axis_index("s") * NC + lax.axis_index("c")

        def row_body(r, carry):
            row = wid * RPW + r
            pltpu.sync_copy(in_hbm.at[row], rowbuf)
            _row_topk(rowbuf, rbuf, obuf)
            pltpu.sync_copy(obuf, out_hbm.at[row])
            return carry

        lax.fori_loop(0, RPW, row_body, 0)

    return sc_topk


_sc_topk = _make_sc_kernel()


@jax.jit
def kernel(input):
    return _sc_topk(input)


# speedup vs baseline: 5.7451x; 1.6461x over previous
import functools

import jax
import jax.numpy as jnp
from jax import lax
from jax.experimental import pallas as pl
from jax.experimental.pallas import tpu as pltpu
from jax.experimental.pallas import tpu_sc as plsc

K = 64
N = 32768
ROWS = 128
L = 16
VPS = 8
SPG = 4
VPG = VPS * SPG
GSZ = VPG * L
G = N // GSZ
NC = 2
NS = 16
NW = NC * NS
RPW = ROWS // NW


def _neg():
    return jnp.full((L,), -jnp.inf, jnp.float32)


def _sortd(v):
    return plsc.sort_key_val(v, v, descending=True)[0]


def _merge2(a, b):
    rb = lax.rev(b, (0,))
    hi = jnp.maximum(a, rb)
    lo = jnp.minimum(a, rb)
    return _sortd(hi), _sortd(lo)


def _insert(s, v):
    s0, s1, s2, s3 = s
    rv = lax.rev(_sortd(v), (0,))
    t3 = _sortd(jnp.maximum(s3, rv))
    s2, t3 = _merge2(s2, t3)
    s1, s2 = _merge2(s1, s2)
    s0, s1 = _merge2(s0, s1)
    return (s0, s1, s2, t3)


def _row_topk(rowbuf, rbuf, sbuf, cbuf):
    z = _neg()

    def p1(g, s):
        base = g * GSZ
        gm = None
        for si in range(SPG):
            m = rowbuf[pl.ds(base + si * VPS * L, L)]
            for j in range(1, VPS):
                m = jnp.maximum(m, rowbuf[pl.ds(base + (si * VPS + j) * L, L)])
            sbuf[pl.ds((g * SPG + si) * L, L)] = m
            gm = m if gm is None else jnp.maximum(gm, m)
        rbuf[pl.ds(g * L, L)] = gm
        return lax.cond(jnp.any(gm > s[3][15]),
                        lambda ss: _insert(ss, gm), lambda ss: ss, s)

    s = lax.fori_loop(0, G, p1, (z, z, z, z))
    t = s[3][15]

    def p2(g, c):
        gmax = rbuf[pl.ds(g * L, L)]

        def scan_group(c):
            for si in range(SPG):
                smax = sbuf[pl.ds((g * SPG + si) * L, L)]

                def scan_sub(cc, si=si):
                    for j in range(VPS):
                        v = rowbuf[pl.ds(g * GSZ + (si * VPS + j) * L, L)]
                        mask = v >= t
                        plsc.store_compressed(cbuf.at[pl.ds(cc, L)], v,
                                              mask=mask)
                        cc = cc + plsc.all_reduce_population_count(mask)[0]
                    return cc

                c = lax.cond(jnp.any(smax >= t), scan_sub, lambda cc: cc, c)
            return c

        return lax.cond(jnp.any(gmax >= t), scan_group, lambda cc: cc, c)

    c = lax.fori_loop(0, G, p2, jnp.int32(0))

    cbuf[pl.ds(c, L)] = z
    nv = (c + (L - 1)) // L

    def fin(i, s):
        return _insert(s, cbuf[pl.ds(i * L, L)])

    return lax.fori_loop(0, nv, fin, (z, z, z, z))


def _make_sc_kernel():
    mesh = plsc.VectorSubcoreMesh(core_axis_name="c", subcore_axis_name="s")

    @functools.partial(
        pl.kernel,
        mesh=mesh,
        out_type=jax.ShapeDtypeStruct((ROWS, K), jnp.float32),
        compiler_params=pltpu.CompilerParams(needs_layout_passes=False),
        scratch_types=[
            pltpu.VMEM((2 * N,), jnp.float32),
            pltpu.VMEM((G * L,), jnp.float32),
            pltpu.VMEM((G * SPG * L,), jnp.float32),
            pltpu.VMEM((N + L,), jnp.float32),
            pltpu.VMEM((RPW, K), jnp.float32),
            pltpu.SemaphoreType.DMA,
            pltpu.SemaphoreType.DMA,
        ],
    )
    def sc_topk(in_hbm, out_hbm, rowbuf2, rbuf, sbuf, cbuf, obuf, sem0, sem1):
        wid = lax.axis_index("s") * NC + lax.axis_index("c")
        row0 = wid * RPW
        sems = (sem0, sem1)

        copies = [None, None]
        copies[0] = pltpu.async_copy(in_hbm.at[row0], rowbuf2.at[pl.ds(0, N)], sem0)
        for r in range(RPW):
            b = r % 2
            copies[b].wait()
            if r + 1 < RPW:
                nb = (r + 1) % 2
                copies[nb] = pltpu.async_copy(
                    in_hbm.at[row0 + r + 1], rowbuf2.at[pl.ds(nb * N, N)], sems[nb])
            s = _row_topk(rowbuf2.at[pl.ds(b * N, N)], rbuf, sbuf, cbuf)
            for blk in range(4):
                obuf[r, pl.ds(blk * L, L)] = s[blk]
        pltpu.sync_copy(obuf, out_hbm.at[pl.ds(row0, RPW)])

    return sc_topk


_sc_topk = _make_sc_kernel()


@jax.jit
def kernel(input):
    return _sc_topk(input)
